# Initial kernel scaffold; baseline (speedup 1.0000x reference)
import jax
import jax.numpy as jnp
from jax.experimental import pallas as pl


def kernel(inputs, emb_matrix):
    raise NotImplementedError("write your pallas kernel here")


# speedup vs baseline: 1.1016x; 1.1016x over previous
import functools

import jax
import jax.numpy as jnp
from jax import lax
from jax.experimental import pallas as pl
from jax.experimental.pallas import tpu as pltpu
from jax.experimental.pallas import tpu_sc as plsc

NC = 2
NS = 16
NW = NC * NS

CHUNK = 1024


@functools.cache
def _make_gather(n_rows: int, emb_dim: int):
    assert n_rows % (NW * CHUNK) == 0
    b_per_w = n_rows // NW
    n_chunks = b_per_w // CHUNK

    mesh = plsc.VectorSubcoreMesh(
        core_axis_name="c", subcore_axis_name="s",
        num_cores=NC, num_subcores=NS,
    )

    @functools.partial(
        pl.kernel,
        out_type=jax.ShapeDtypeStruct((n_rows, emb_dim), jnp.float32),
        mesh=mesh,
        scratch_types=[
            pltpu.VMEM((CHUNK,), jnp.int32),
            pltpu.VMEM((CHUNK, emb_dim), jnp.float32),
            pltpu.SemaphoreType.DMA,
        ],
        compiler_params=pltpu.CompilerParams(use_tc_tiling_on_sc=False),
    )
    def gather_kernel(idx_hbm, table_hbm, out_hbm, idx_v, rows_v, sem):
        wid = lax.axis_index("s") * NC + lax.axis_index("c")
        base = wid * b_per_w

        def step(g, carry):
            off = base + g * CHUNK
            pltpu.sync_copy(idx_hbm.at[pl.ds(off, CHUNK)], idx_v)
            pltpu.async_copy(table_hbm.at[idx_v], rows_v, sem).wait()
            pltpu.sync_copy(rows_v, out_hbm.at[pl.ds(off, CHUNK)])
            return carry

        lax.fori_loop(0, n_chunks, step, 0)

    return gather_kernel


def kernel(inputs, emb_matrix):
    batch, fields = inputs.shape
    emb_dim = emb_matrix.shape[1]
    flat_idx = inputs.reshape(-1).astype(jnp.int32)
    out = _make_gather(flat_idx.shape[0], emb_dim)(flat_idx, emb_matrix)
    return out.reshape(batch, fields, emb_dim)


# speedup vs baseline: 1.1129x; 1.0103x over previous
import functools

import jax
import jax.numpy as jnp
from jax import lax
from jax.experimental import pallas as pl
from jax.experimental.pallas import tpu as pltpu
from jax.experimental.pallas import tpu_sc as plsc

NC = 2
NS = 16
NW = NC * NS

CHUNK = 512
NBUF = 4


@functools.cache
def _make_gather(n_rows: int, emb_dim: int):
    assert n_rows % (NW * CHUNK) == 0
    b_per_w = n_rows // NW
    n_chunks = b_per_w // CHUNK
    assert n_chunks >= NBUF

    mesh = plsc.VectorSubcoreMesh(
        core_axis_name="c", subcore_axis_name="s",
        num_cores=NC, num_subcores=NS,
    )

    @functools.partial(
        pl.kernel,
        out_type=jax.ShapeDtypeStruct((n_rows, emb_dim), jnp.float32),
        mesh=mesh,
        scratch_types=[
            pltpu.VMEM((b_per_w,), jnp.int32),
            [pltpu.VMEM((CHUNK, emb_dim), jnp.float32) for _ in range(NBUF)],
            [pltpu.SemaphoreType.DMA for _ in range(NBUF)],
            [pltpu.SemaphoreType.DMA for _ in range(NBUF)],
        ],
        compiler_params=pltpu.CompilerParams(use_tc_tiling_on_sc=False),
    )
    def gather_kernel(idx_hbm, table_hbm, out_hbm, idx_all, rows, sg, ss):
        wid = lax.axis_index("s") * NC + lax.axis_index("c")
        base = wid * b_per_w

        pltpu.sync_copy(idx_hbm.at[pl.ds(base, b_per_w)], idx_all)

        def gather(g, b):
            pltpu.async_copy(
                table_hbm.at[idx_all.at[pl.ds(g * CHUNK, CHUNK)]],
                rows[b], sg[b])

        def store(g, b):
            pltpu.async_copy(
                rows[b], out_hbm.at[pl.ds(base + g * CHUNK, CHUNK)], ss[b])

        for b in range(NBUF):
            gather(b, b)

        def step(i, carry):
            for b in range(NBUF):
                g = i * NBUF + b
                pltpu.make_async_copy(
                    table_hbm.at[idx_all.at[pl.ds(0, CHUNK)]],
                    rows[b], sg[b]).wait()
                store(g, b)

                @pl.when(g + NBUF < n_chunks)
                def _():
                    pltpu.make_async_copy(
                        rows[b],
                        out_hbm.at[pl.ds(base, CHUNK)], ss[b]).wait()
                    gather(g + NBUF, b)
            return carry

        lax.fori_loop(0, n_chunks // NBUF, step, 0)

        for b in range(NBUF):
            pltpu.make_async_copy(
                rows[b], out_hbm.at[pl.ds(base, CHUNK)], ss[b]).wait()

    return gather_kernel


def kernel(inputs, emb_matrix):
    batch, fields = inputs.shape
    emb_dim = emb_matrix.shape[1]
    flat_idx = inputs.reshape(-1).astype(jnp.int32)
    out = _make_gather(flat_idx.shape[0], emb_dim)(flat_idx, emb_matrix)
    return out.reshape(batch, fields, emb_dim)


# speedup vs baseline: 4.4354x; 3.9854x over previous
import functools

import jax
import jax.numpy as jnp
from jax import lax
from jax.experimental import pallas as pl
from jax.experimental.pallas import tpu as pltpu
from jax.experimental.pallas import tpu_sc as plsc

NC = 2
NS = 16
NW = NC * NS

NBUF = 8


@functools.cache
def _make_gather(batch: int, fields: int, emb_dim: int):
    assert batch % NW == 0
    rows_per_w = batch // NW
    assert rows_per_w % NBUF == 0

    mesh = plsc.VectorSubcoreMesh(
        core_axis_name="c", subcore_axis_name="s",
        num_cores=NC, num_subcores=NS,
    )

    @functools.partial(
        pl.kernel,
        out_type=jax.ShapeDtypeStruct((batch, fields, emb_dim), jnp.float32),
        mesh=mesh,
        scratch_types=[
            pltpu.VMEM((rows_per_w, fields), jnp.int32),
            [pltpu.VMEM((fields, emb_dim), jnp.float32) for _ in range(NBUF)],
            [pltpu.SemaphoreType.DMA for _ in range(NBUF)],
            [pltpu.SemaphoreType.DMA for _ in range(NBUF)],
        ],
        compiler_params=pltpu.CompilerParams(use_tc_tiling_on_sc=False),
    )
    def gather_kernel(idx_hbm, table_hbm, out_hbm, idx_all, rows, sg, ss):
        wid = lax.axis_index("s") * NC + lax.axis_index("c")
        base = wid * rows_per_w

        pltpu.sync_copy(idx_hbm.at[pl.ds(base, rows_per_w), :], idx_all)

        def gather(r, b):
            pltpu.async_copy(table_hbm.at[idx_all.at[r, :]], rows[b], sg[b])

        def store(r, b):
            pltpu.async_copy(rows[b], out_hbm.at[base + r], ss[b])

        def wait_gather(b):
            pltpu.make_async_copy(
                table_hbm.at[idx_all.at[0, :]], rows[b], sg[b]).wait()

        def wait_store(b):
            pltpu.make_async_copy(rows[b], out_hbm.at[base], ss[b]).wait()

        for b in range(NBUF):
            gather(b, b)

        def step(i, carry):
            for b in range(NBUF):
                r = i * NBUF + b
                wait_gather(b)
                store(r, b)

                @pl.when(r + NBUF < rows_per_w)
                def _():
                    wait_store(b)
                    gather(r + NBUF, b)
            return carry

        lax.fori_loop(0, rows_per_w // NBUF, step, 0)

        for b in range(NBUF):
            wait_store(b)

    return gather_kernel


def kernel(inputs, emb_matrix):
    batch, fields = inputs.shape
    emb_dim = emb_matrix.shape[1]
    return _make_gather(batch, fields, emb_dim)(inputs, emb_matrix)
